# baseline (speedup 1.0000x reference)
import jax
import jax.numpy as jnp
from jax import lax
from jax.experimental import pallas as pl
from jax.experimental.pallas import tpu as pltpu, tpu_sc as plsc

NUM_DOMAINS = 5
EMBED_DIM = 16
BATCH = 16384
L = 16

NC = 2
NS = 16
NW = NC * NS
B_PER_W = BATCH // NW
N_BLOCKS = B_PER_W // L
BLK_ELEMS = L * EMBED_DIM

N_CHUNKS = 4
BLK_PER_CHUNK = N_BLOCKS // N_CHUNKS
ELEM_PER_CHUNK = BLK_PER_CHUNK * BLK_ELEMS

_mesh = plsc.VectorSubcoreMesh(core_axis_name="c", subcore_axis_name="s")


def _body(ids_hbm, table_hbm, out_hbm, idx_v, tab_v, rows_v, sem):
    wid = lax.axis_index("s") * NC + lax.axis_index("c")
    base = wid * B_PER_W
    c_tab = pltpu.async_copy(table_hbm, tab_v, sem)
    c_ids = pltpu.async_copy(ids_hbm.at[pl.ds(base, B_PER_W)], idx_v, sem)
    c_tab.wait()
    c_ids.wait()
    iota = lax.iota(jnp.int32, L)
    iota16 = iota * EMBED_DIM
    rots = [(iota + d) & (L - 1) for d in range(EMBED_DIM)]
    soffs = [iota16 + r for r in rots]

    def block(b):
        v_ids16 = idx_v[pl.ds(b * L, L)] * EMBED_DIM
        blk = rows_v.at[pl.ds(b * BLK_ELEMS, BLK_ELEMS)]
        for d in range(EMBED_DIM):
            vals = plsc.load_gather(tab_v, [v_ids16 + rots[d]])
            plsc.store_scatter(blk, [soffs[d]], vals)

    pend = []
    for c in range(N_CHUNKS):
        for b in range(c * BLK_PER_CHUNK, (c + 1) * BLK_PER_CHUNK):
            block(b)
        e0 = c * ELEM_PER_CHUNK
        pend.append(
            pltpu.async_copy(
                rows_v.at[pl.ds(e0, ELEM_PER_CHUNK)],
                out_hbm.at[pl.ds(base * EMBED_DIM + e0, ELEM_PER_CHUNK)],
                sem,
            )
        )
    for p in pend:
        p.wait()


_sc_lookup = pl.kernel(
    _body,
    out_type=jax.ShapeDtypeStruct((BATCH * EMBED_DIM,), jnp.float32),
    mesh=_mesh,
    scratch_types=[
        pltpu.VMEM((B_PER_W,), jnp.int32),
        pltpu.VMEM((NUM_DOMAINS * EMBED_DIM,), jnp.float32),
        pltpu.VMEM((B_PER_W * EMBED_DIM,), jnp.float32),
        pltpu.SemaphoreType.DMA,
    ],
    compiler_params=pltpu.CompilerParams(
        use_tc_tiling_on_sc=True, needs_layout_passes=False
    ),
)


@jax.jit
def kernel(domain_ids, table):
    flat = _sc_lookup(domain_ids.astype(jnp.int32), table.reshape(-1))
    return flat.reshape(BATCH, EMBED_DIM)


# speedup vs baseline: 1.0787x; 1.0787x over previous
import jax
import jax.numpy as jnp
from jax import lax
from jax.experimental import pallas as pl
from jax.experimental.pallas import tpu as pltpu, tpu_sc as plsc

NUM_DOMAINS = 5
EMBED_DIM = 16
BATCH = 16384
L = 16

NC = 2
NS = 16
NW = NC * NS
B_PER_W = BATCH // NW
N_BLOCKS = B_PER_W // L
BLK_ELEMS = L * EMBED_DIM

N_CHUNKS = 4
BLK_PER_CHUNK = N_BLOCKS // N_CHUNKS
ELEM_PER_CHUNK = BLK_PER_CHUNK * BLK_ELEMS

_mesh = plsc.VectorSubcoreMesh(core_axis_name="c", subcore_axis_name="s")


def _body(ids_hbm, table_hbm, out_hbm, idx_v, tab_v, rows_v, sem):
    wid = lax.axis_index("s") * NC + lax.axis_index("c")
    base = wid * B_PER_W
    c_tab = pltpu.async_copy(table_hbm, tab_v, sem)
    c_ids = pltpu.async_copy(ids_hbm.at[pl.ds(base, B_PER_W)], idx_v, sem)
    c_tab.wait()
    c_ids.wait()
    iota = lax.iota(jnp.int32, L)
    iota16 = iota * EMBED_DIM
    rots = [(iota + d) & (L - 1) for d in range(EMBED_DIM)]
    soffs = [iota16 + r for r in rots]

    def block(b):
        v_ids16 = idx_v[pl.ds(b * L, L)] * EMBED_DIM
        blk = rows_v.at[pl.ds(b * BLK_ELEMS, BLK_ELEMS)]
        for d in range(EMBED_DIM):
            vals = plsc.load_gather(tab_v, [v_ids16 + rots[d]])
            plsc.store_scatter(blk, [soffs[d]], vals)

    def step(s, _):
        block(2 * s)
        block(2 * s + 1)
        return 0

    pend = []
    for c in range(N_CHUNKS):
        lax.fori_loop(
            c * BLK_PER_CHUNK // 2, (c + 1) * BLK_PER_CHUNK // 2, step, 0
        )
        e0 = c * ELEM_PER_CHUNK
        pend.append(
            pltpu.async_copy(
                rows_v.at[pl.ds(e0, ELEM_PER_CHUNK)],
                out_hbm.at[pl.ds(base * EMBED_DIM + e0, ELEM_PER_CHUNK)],
                sem,
            )
        )
    for p in pend:
        p.wait()


_sc_lookup = pl.kernel(
    _body,
    out_type=jax.ShapeDtypeStruct((BATCH * EMBED_DIM,), jnp.float32),
    mesh=_mesh,
    scratch_types=[
        pltpu.VMEM((B_PER_W,), jnp.int32),
        pltpu.VMEM((NUM_DOMAINS * EMBED_DIM,), jnp.float32),
        pltpu.VMEM((B_PER_W * EMBED_DIM,), jnp.float32),
        pltpu.SemaphoreType.DMA,
    ],
    compiler_params=pltpu.CompilerParams(
        use_tc_tiling_on_sc=True, needs_layout_passes=False
    ),
)


@jax.jit
def kernel(domain_ids, table):
    flat = _sc_lookup(domain_ids.astype(jnp.int32), table.reshape(-1))
    return flat.reshape(BATCH, EMBED_DIM)
